# baseline (speedup 1.0000x reference)
import functools

import jax
import jax.numpy as jnp
from jax.experimental import pallas as pl
from jax.experimental.pallas import tpu as pltpu

_BLK = 2048


def _body(x_ref, emb_ref, w_ref, b_ref, o_ref):
    xb = x_ref[...]
    x0 = xb[:, 0:1]
    x1 = xb[:, 1:2]
    e0 = emb_ref[0, 0]
    e1 = emb_ref[0, 1]
    e = jnp.where(x1 >= 1.0, e1, e0)
    o_ref[...] = x0 * w_ref[0:1, :] + e * w_ref[1:2, :] + b_ref[...]


@jax.jit
def _run(x, emb_row, w_t, b_row):
    B = x.shape[0]
    N = w_t.shape[1]
    grid = (B // _BLK,)
    return pl.pallas_call(
        _body,
        grid=grid,
        in_specs=[
            pl.BlockSpec((_BLK, 2), lambda i: (i, 0)),
            pl.BlockSpec((1, 2), lambda i: (0, 0)),
            pl.BlockSpec((2, N), lambda i: (0, 0)),
            pl.BlockSpec((1, N), lambda i: (0, 0)),
        ],
        out_specs=pl.BlockSpec((_BLK, N), lambda i: (i, 0)),
        out_shape=jax.ShapeDtypeStruct((B, N), jnp.float32),
    )(x, emb_row, w_t, b_row)


def kernel(x, emb16, fc1_w, fc1_b):
    emb_row = emb16.reshape(1, 2)
    w_t = fc1_w.T
    b_row = fc1_b.reshape(1, -1)
    return _run(x, emb_row, w_t, b_row)
